# baseline (speedup 1.0000x reference)
import functools

import jax
import jax.numpy as jnp
from jax import lax
from jax.experimental import pallas as pl
from jax.experimental.pallas import tpu as pltpu
from jax.experimental.pallas import tpu_sc as plsc

NC, NS, LN = 2, 16, 16
NW = NC * NS
F = 512
K = 8
B = 1024
C = 4
FC = F * C
J = F * K
JC = J * C
B_PER_W = B // NW


def _d2_topk_body(crd_ref, idx_ref):
    x = crd_ref[...]
    xb = x.astype(jnp.bfloat16)
    g = lax.dot_general(xb, xb, (((0,), (0,)), ((), ())),
                        preferred_element_type=jnp.float32)
    xx = jnp.sum(jnp.square(x), axis=0)
    d = g * -2.0
    d = d + xx[None, :]
    d = d + xx[:, None]
    d = jnp.maximum(d, 0.0)
    d = jnp.sqrt(d)

    iota_j = lax.broadcasted_iota(jnp.int32, (F, F), 1)
    rows = []
    for _ in range(K):
        mn = jnp.min(d, axis=1, keepdims=True)
        cand = jnp.where(d == mn, iota_j, jnp.int32(F))
        win = jnp.min(cand, axis=1)
        rows.append(win)
        d = jnp.where(iota_j == win[:, None], jnp.inf, d)
    idxm = jnp.concatenate([w[None, :] for w in rows], axis=0)
    rr = lax.broadcasted_iota(jnp.int32, (K, F), 0)
    cc = lax.broadcasted_iota(jnp.int32, (K, F), 1)
    idx_ref[...] = jnp.where((rr == 0) & (cc == 0), 0, idxm)


def _d2_topk(crd):
    return pl.pallas_call(
        _d2_topk_body,
        out_shape=jax.ShapeDtypeStruct((K, F), jnp.int32),
    )(crd)


NB = 2
NGRP = B_PER_W // NB
NCHUNK = 1
CHUNK = JC // NCHUNK
CV = CHUNK // LN


def _gather_body(in_hbm, idx_hbm, out_hbm, idx_v, a_v, in_v, out_v,
                 sem_in, sem_out):
    wid = lax.axis_index("c") * NS + lax.axis_index("s")
    base = wid * B_PER_W
    iota = lax.iota(jnp.int32, LN)
    pltpu.sync_copy(idx_hbm, idx_v)

    def fire_in(p, buf):
        cps = [pltpu.make_async_copy(
                   in_hbm.at[base + NB * p + k],
                   in_v.at[pl.ds((buf * NB + k) * FC, FC)], sem_in)
               for k in range(NB)]
        for cp in cps:
            cp.start()
        return cps

    in_cp = {0: fire_in(0, 0)}

    @plsc.parallel_loop(0, JC // LN, 1, unroll=8)
    def _(g):
        j = ((g >> 5) << 7) + ((g & 7) << 4) + iota
        f = j >> 3
        jidx = plsc.load_gather(
            idx_v, [((f >> 7) << 10) + ((j & 7) << 7) + (f & 127)])
        c = (g >> 3) & 3
        a_v[pl.ds(g * LN, LN)] = \
            ((jidx >> 7) << 9) + c * 128 + (jidx & 127)

    out_cp = {}
    step = 0
    for p in range(NGRP):
        ibuf = p & 1
        if p + 1 < NGRP:
            in_cp[p + 1] = fire_in(p + 1, ibuf ^ 1)
        for cp in in_cp.pop(p):
            cp.wait()
        for h in range(NCHUNK):
            obuf = step & 1
            if step >= 2:
                for cp in out_cp.pop(step - 2):
                    cp.wait()

            @plsc.parallel_loop(0, CV, 1, unroll=8)
            def _(g):
                a = a_v[pl.ds(h * CHUNK + g * LN, LN)]
                for k in range(NB):
                    out_v[pl.ds((obuf * NB + k) * CHUNK + g * LN, LN)] = \
                        plsc.load_gather(in_v, [a + (ibuf * NB + k) * FC])
            cps = [pltpu.make_async_copy(
                       out_v.at[pl.ds((obuf * NB + k) * CHUNK, CHUNK)],
                       out_hbm.at[base + NB * p + k,
                                  pl.ds(h * CHUNK, CHUNK)], sem_out)
                   for k in range(NB)]
            for cp in cps:
                cp.start()
            out_cp[step] = cps
            step += 1
    for s in sorted(out_cp):
        for cp in out_cp[s]:
            cp.wait()


def _gather(in2, idx):
    mesh = plsc.VectorSubcoreMesh(core_axis_name="c", subcore_axis_name="s",
                                  num_cores=NC, num_subcores=NS)
    kern = functools.partial(
        pl.kernel,
        out_type=jax.ShapeDtypeStruct((B, JC), jnp.float32),
        mesh=mesh,
        compiler_params=pltpu.CompilerParams(needs_layout_passes=False,
                                             use_tc_tiling_on_sc=False),
        scratch_types=[
            pltpu.VMEM((J,), jnp.int32),
            pltpu.VMEM((JC,), jnp.int32),
            pltpu.VMEM((2 * NB * FC,), jnp.float32),
            pltpu.VMEM((2 * NB * CHUNK,), jnp.float32),
            pltpu.SemaphoreType.DMA,
            pltpu.SemaphoreType.DMA,
        ],
    )(_gather_body)
    return kern(in2, idx)


def kernel(coordinates, inputs):
    crd = coordinates.reshape(coordinates.shape[0], F)
    idxm = _d2_topk(crd)
    idx = idxm.reshape(K, F // 128, 128).transpose(1, 0, 2).reshape(J)
    in2 = (inputs.reshape(B, F // 128, 128, C)
           .transpose(0, 1, 3, 2).reshape(B, FC))
    out = _gather(in2, idx)
    return (out.reshape(B, J // 128, C, 128)
            .transpose(0, 1, 3, 2).reshape(B, 1, J, C))


# speedup vs baseline: 1.0005x; 1.0005x over previous
import functools

import jax
import jax.numpy as jnp
from jax import lax
from jax.experimental import pallas as pl
from jax.experimental.pallas import tpu as pltpu
from jax.experimental.pallas import tpu_sc as plsc

NC, NS, LN = 2, 16, 16
NW = NC * NS
F = 512
K = 8
B = 1024
C = 4
FC = F * C
J = F * K
JC = J * C
B_PER_W = B // NW


def _d2_topk_body(crd_ref, idx_ref):
    x = crd_ref[...]
    xb = x.astype(jnp.bfloat16)
    g = lax.dot_general(xb, xb, (((0,), (0,)), ((), ())),
                        preferred_element_type=jnp.float32)
    xx = jnp.sum(jnp.square(x), axis=0)
    d = g * -2.0
    d = d + xx[None, :]
    d = d + xx[:, None]
    d = jnp.maximum(d, 0.0)
    d = jnp.sqrt(d)

    iota_j = lax.broadcasted_iota(jnp.int32, (F, F), 1)
    rows = []
    for _ in range(K):
        mn = jnp.min(d, axis=1, keepdims=True)
        cand = jnp.where(d == mn, iota_j, jnp.int32(F))
        win = jnp.min(cand, axis=1)
        rows.append(win)
        d = jnp.where(iota_j == win[:, None], jnp.inf, d)
    idxm = jnp.concatenate([w[None, :] for w in rows], axis=0)
    rr = lax.broadcasted_iota(jnp.int32, (K, F), 0)
    cc = lax.broadcasted_iota(jnp.int32, (K, F), 1)
    idx_ref[...] = jnp.where((rr == 0) & (cc == 0), 0, idxm)


def _d2_topk(crd):
    return pl.pallas_call(
        _d2_topk_body,
        out_shape=jax.ShapeDtypeStruct((K, F), jnp.int32),
    )(crd)


NB = 4
NGRP = B_PER_W // NB
NCHUNK = 2
CHUNK = JC // NCHUNK
CV = CHUNK // LN


def _gather_body(in_hbm, idx_hbm, out_hbm, idx_v, a_v, in_v, out_v,
                 sem_in, sem_out):
    wid = lax.axis_index("c") * NS + lax.axis_index("s")
    base = wid * B_PER_W
    iota = lax.iota(jnp.int32, LN)
    pltpu.sync_copy(idx_hbm, idx_v)

    def fire_in(p, buf):
        cps = [pltpu.make_async_copy(
                   in_hbm.at[base + NB * p + k],
                   in_v.at[pl.ds((buf * NB + k) * FC, FC)], sem_in)
               for k in range(NB)]
        for cp in cps:
            cp.start()
        return cps

    in_cp = {0: fire_in(0, 0)}

    @plsc.parallel_loop(0, JC // LN, 1, unroll=8)
    def _(g):
        j = ((g >> 5) << 7) + ((g & 7) << 4) + iota
        f = j >> 3
        jidx = plsc.load_gather(
            idx_v, [((f >> 7) << 10) + ((j & 7) << 7) + (f & 127)])
        c = (g >> 3) & 3
        a_v[pl.ds(g * LN, LN)] = \
            ((jidx >> 7) << 9) + c * 128 + (jidx & 127)

    out_cp = {}
    step = 0
    for p in range(NGRP):
        ibuf = p & 1
        if p + 1 < NGRP:
            in_cp[p + 1] = fire_in(p + 1, ibuf ^ 1)
        for cp in in_cp.pop(p):
            cp.wait()
        for h in range(NCHUNK):
            obuf = step & 1
            if step >= 2:
                for cp in out_cp.pop(step - 2):
                    cp.wait()

            @plsc.parallel_loop(0, CV, 1, unroll=16)
            def _(g):
                a = a_v[pl.ds(h * CHUNK + g * LN, LN)]
                for k in range(NB):
                    out_v[pl.ds((obuf * NB + k) * CHUNK + g * LN, LN)] = \
                        plsc.load_gather(in_v, [a + (ibuf * NB + k) * FC])
            cps = [pltpu.make_async_copy(
                       out_v.at[pl.ds((obuf * NB + k) * CHUNK, CHUNK)],
                       out_hbm.at[base + NB * p + k,
                                  pl.ds(h * CHUNK, CHUNK)], sem_out)
                   for k in range(NB)]
            for cp in cps:
                cp.start()
            out_cp[step] = cps
            step += 1
    for s in sorted(out_cp):
        for cp in out_cp[s]:
            cp.wait()


def _gather(in2, idx):
    mesh = plsc.VectorSubcoreMesh(core_axis_name="c", subcore_axis_name="s",
                                  num_cores=NC, num_subcores=NS)
    kern = functools.partial(
        pl.kernel,
        out_type=jax.ShapeDtypeStruct((B, JC), jnp.float32),
        mesh=mesh,
        compiler_params=pltpu.CompilerParams(needs_layout_passes=False,
                                             use_tc_tiling_on_sc=False),
        scratch_types=[
            pltpu.VMEM((J,), jnp.int32),
            pltpu.VMEM((JC,), jnp.int32),
            pltpu.VMEM((2 * NB * FC,), jnp.float32),
            pltpu.VMEM((2 * NB * CHUNK,), jnp.float32),
            pltpu.SemaphoreType.DMA,
            pltpu.SemaphoreType.DMA,
        ],
    )(_gather_body)
    return kern(in2, idx)


def kernel(coordinates, inputs):
    crd = coordinates.reshape(coordinates.shape[0], F)
    idxm = _d2_topk(crd)
    idx = idxm.reshape(K, F // 128, 128).transpose(1, 0, 2).reshape(J)
    in2 = (inputs.reshape(B, F // 128, 128, C)
           .transpose(0, 1, 3, 2).reshape(B, FC))
    out = _gather(in2, idx)
    return (out.reshape(B, J // 128, C, 128)
            .transpose(0, 1, 3, 2).reshape(B, 1, J, C))


# speedup vs baseline: 1.0252x; 1.0247x over previous
import functools

import jax
import jax.numpy as jnp
from jax import lax
from jax.experimental import pallas as pl
from jax.experimental.pallas import tpu as pltpu
from jax.experimental.pallas import tpu_sc as plsc

NC, NS, LN = 2, 16, 16
NW = NC * NS
F = 512
K = 8
B = 1024
C = 4
FC = F * C
J = F * K
JC = J * C
B_PER_W = B // NW


def _d2_topk_body(crd_ref, idx_ref):
    x = crd_ref[...]
    xb = x.astype(jnp.bfloat16)
    g = lax.dot_general(xb, xb, (((0,), (0,)), ((), ())),
                        preferred_element_type=jnp.float32)
    xx = jnp.sum(jnp.square(x), axis=0)
    d = g * -2.0
    d = d + xx[None, :]
    d = d + xx[:, None]
    d = jnp.maximum(d, 0.0)
    d = jnp.sqrt(d)

    iota_j = lax.broadcasted_iota(jnp.int32, (F, F), 1)
    rows = []
    for _ in range(K):
        mn = jnp.min(d, axis=1, keepdims=True)
        cand = jnp.where(d == mn, iota_j, jnp.int32(F))
        win = jnp.min(cand, axis=1)
        rows.append(win)
        d = jnp.where(iota_j == win[:, None], jnp.inf, d)
    idxm = jnp.concatenate([w[None, :] for w in rows], axis=0)
    rr = lax.broadcasted_iota(jnp.int32, (K, F), 0)
    cc = lax.broadcasted_iota(jnp.int32, (K, F), 1)
    idx_ref[...] = jnp.where((rr == 0) & (cc == 0), 0, idxm)


def _d2_topk(crd):
    return pl.pallas_call(
        _d2_topk_body,
        out_shape=jax.ShapeDtypeStruct((K, F), jnp.int32),
    )(crd)


NB = 4
NGRP = B_PER_W // NB
NCHUNK = 2
CHUNK = JC // NCHUNK
CV = CHUNK // LN


def _gather_body(in_hbm, idx_hbm, out_hbm, idx_v, a_v, in_v, out_v,
                 sem_in, sem_out):
    wid = lax.axis_index("c") * NS + lax.axis_index("s")
    base = wid * B_PER_W
    iota = lax.iota(jnp.int32, LN)
    pltpu.sync_copy(idx_hbm, idx_v)

    def fire_in(p, buf):
        cps = [pltpu.make_async_copy(
                   in_hbm.at[base + NB * p + k],
                   in_v.at[pl.ds((buf * NB + k) * FC, FC)], sem_in)
               for k in range(NB)]
        for cp in cps:
            cp.start()
        return cps

    in_cp = {0: fire_in(0, 0)}

    @plsc.parallel_loop(0, JC // LN, 1, unroll=8)
    def _(g):
        j = ((g >> 5) << 7) + ((g & 7) << 4) + iota
        f = j >> 3
        jidx = plsc.load_gather(
            idx_v, [((f >> 7) << 10) + ((j & 7) << 7) + (f & 127)])
        c = (g >> 3) & 3
        a_v[pl.ds(g * LN, LN)] = \
            ((jidx >> 7) << 9) + c * 128 + (jidx & 127)

    out_cp = {}
    step = 0
    for p in range(NGRP):
        ibuf = p & 1
        if p + 1 < NGRP:
            in_cp[p + 1] = fire_in(p + 1, ibuf ^ 1)
        for cp in in_cp.pop(p):
            cp.wait()
        for h in range(NCHUNK):
            obuf = step & 1
            if step >= 2:
                for cp in out_cp.pop(step - 2):
                    cp.wait()

            @plsc.parallel_loop(0, CV, 1, unroll=8)
            def _(g):
                a = a_v[pl.ds(h * CHUNK + g * LN, LN)]
                for k in range(NB):
                    out_v[pl.ds((obuf * NB + k) * CHUNK + g * LN, LN)] = \
                        plsc.load_gather(in_v, [a + (ibuf * NB + k) * FC])
            cps = [pltpu.make_async_copy(
                       out_v.at[pl.ds((obuf * NB + k) * CHUNK, CHUNK)],
                       out_hbm.at[base + NB * p + k,
                                  pl.ds(h * CHUNK, CHUNK)], sem_out)
                   for k in range(NB)]
            for cp in cps:
                cp.start()
            out_cp[step] = cps
            step += 1
    for s in sorted(out_cp):
        for cp in out_cp[s]:
            cp.wait()


def _gather(in2, idx):
    mesh = plsc.VectorSubcoreMesh(core_axis_name="c", subcore_axis_name="s",
                                  num_cores=NC, num_subcores=NS)
    kern = functools.partial(
        pl.kernel,
        out_type=jax.ShapeDtypeStruct((B, JC), jnp.float32),
        mesh=mesh,
        compiler_params=pltpu.CompilerParams(needs_layout_passes=False,
                                             use_tc_tiling_on_sc=False),
        scratch_types=[
            pltpu.VMEM((J,), jnp.int32),
            pltpu.VMEM((JC,), jnp.int32),
            pltpu.VMEM((2 * NB * FC,), jnp.float32),
            pltpu.VMEM((2 * NB * CHUNK,), jnp.float32),
            pltpu.SemaphoreType.DMA,
            pltpu.SemaphoreType.DMA,
        ],
    )(_gather_body)
    return kern(in2, idx)


def kernel(coordinates, inputs):
    crd = coordinates.reshape(coordinates.shape[0], F)
    idxm = _d2_topk(crd)
    idx = idxm.reshape(K, F // 128, 128).transpose(1, 0, 2).reshape(J)
    in2 = (inputs.reshape(B, F // 128, 128, C)
           .transpose(0, 1, 3, 2).reshape(B, FC))
    out = _gather(in2, idx)
    return (out.reshape(B, J // 128, C, 128)
            .transpose(0, 1, 3, 2).reshape(B, 1, J, C))
